# Initial kernel scaffold; baseline (speedup 1.0000x reference)
import functools

import jax
import jax.numpy as jnp
from jax import lax
from jax.experimental import pallas as pl
from jax.experimental.pallas import tpu as pltpu
from jax.experimental.pallas import tpu_sc as plsc

N = 10000
E = 320000
D_FEAT = 128
D_EDGE = 16
MSG = 64
HID = 128

NC = 2
NS = 16
NW = NC * NS
CHUNK = 128
K_CHUNKS = 79
EPW = CHUNK * K_CHUNKS
E_PAD = NW * EPW
N_PAD = 10016
EBLK = 2048


def _node_proj_body(h_ref, ws_ref, wd_ref, a_ref, b_ref):
    h = h_ref[...]
    a_ref[...] = jnp.dot(h, ws_ref[...], preferred_element_type=jnp.float32)
    b_ref[...] = jnp.dot(h, wd_ref[...], preferred_element_type=jnp.float32)


_node_proj = pl.pallas_call(
    _node_proj_body,
    out_shape=[
        jax.ShapeDtypeStruct((N_PAD, MSG), jnp.float32),
        jax.ShapeDtypeStruct((N_PAD, MSG), jnp.float32),
    ],
)


def _edge_proj_body(e_ref, w_ref, b_ref, c_ref):
    c_ref[...] = (
        jnp.dot(e_ref[...], w_ref[...], preferred_element_type=jnp.float32)
        + b_ref[...]
    )


_edge_proj = pl.pallas_call(
    _edge_proj_body,
    grid=(E_PAD // EBLK,),
    in_specs=[
        pl.BlockSpec((EBLK, D_EDGE), lambda i: (i, 0)),
        pl.BlockSpec((D_EDGE, MSG), lambda i: (0, 0)),
        pl.BlockSpec((1, MSG), lambda i: (0, 0)),
    ],
    out_specs=pl.BlockSpec((EBLK, MSG), lambda i: (i, 0)),
    out_shape=jax.ShapeDtypeStruct((E_PAD, MSG), jnp.float32),
)


def _update_body(p_ref, h_ref, wt_ref, wb_ref, bh_ref, o_ref):
    m_sum = p_ref[0] + p_ref[1]
    o_ref[...] = jnp.maximum(
        jnp.dot(m_sum, wt_ref[...], preferred_element_type=jnp.float32)
        + jnp.dot(h_ref[...], wb_ref[...], preferred_element_type=jnp.float32)
        + bh_ref[...],
        0.0,
    )


_update = pl.pallas_call(
    _update_body,
    out_shape=jax.ShapeDtypeStruct((N, HID), jnp.float32),
)


_mesh = plsc.VectorSubcoreMesh(core_axis_name="c", subcore_axis_name="s")


@functools.partial(
    pl.kernel,
    out_type=jax.ShapeDtypeStruct((NC, N, MSG), jnp.float32),
    mesh=_mesh,
    scratch_types=[
        pltpu.VMEM((K_CHUNKS, CHUNK), jnp.int32),
        pltpu.VMEM((K_CHUNKS, CHUNK), jnp.int32),
        pltpu.VMEM((CHUNK, MSG), jnp.float32),
        pltpu.VMEM((CHUNK, MSG), jnp.float32),
        pltpu.VMEM((CHUNK, MSG), jnp.float32),
        pltpu.VMEM((626, MSG), jnp.float32),
        pltpu.VMEM_SHARED((N_PAD, MSG), jnp.float32),
        pltpu.SemaphoreType.DMA,
        pltpu.SemaphoreType.DMA,
        pltpu.SemaphoreType.DMA,
    ],
)
def _edge_phase(a_hbm, b_hbm, c_hbm, src_hbm, dst_hbm, out_hbm,
                src_v, dst_v, a_v, b_v, m_v, z_v, acc, sem_a, sem_b, sem_c):
    cid = lax.axis_index("c")
    sid = lax.axis_index("s")
    wid = sid * NC + cid

    zero16 = jnp.zeros((16,), jnp.float32)

    def _zrow(r, _):
        for j in range(MSG // 16):
            z_v[r, pl.ds(j * 16, 16)] = zero16
        return 0

    lax.fori_loop(0, 626, _zrow, 0)
    pltpu.sync_copy(z_v, acc.at[pl.ds(sid * 626, 626)])
    plsc.subcore_barrier()

    pltpu.sync_copy(src_hbm.at[wid], src_v)
    pltpu.sync_copy(dst_hbm.at[wid], dst_v)
    base = wid * EPW

    def _chunk(k, _):
        cp_a = pltpu.async_copy(a_hbm.at[src_v.at[k]], a_v, sem_a)
        cp_b = pltpu.async_copy(b_hbm.at[dst_v.at[k]], b_v, sem_b)
        cp_c = pltpu.async_copy(
            c_hbm.at[pl.ds(base + k * CHUNK, CHUNK)], m_v, sem_c)
        cp_a.wait()
        cp_b.wait()
        cp_c.wait()

        def _row(r, _):
            for j in range(MSG // 16):
                sl = pl.ds(j * 16, 16)
                m_v[r, sl] = jnp.maximum(a_v[r, sl] + b_v[r, sl] + m_v[r, sl],
                                         0.0)
            return 0

        lax.fori_loop(0, CHUNK, _row, 0)
        pltpu.sync_copy(m_v, acc.at[dst_v.at[k]], add=True)
        return 0

    lax.fori_loop(0, K_CHUNKS, _chunk, 0)
    plsc.subcore_barrier()

    pltpu.sync_copy(acc.at[pl.ds(sid * 625, 625)],
                    out_hbm.at[cid, pl.ds(sid * 625, 625)])


def kernel(x, edge_index, edge_attr, node_ids,
           Wm0, bm0, Wh0, bh0, Wm1, bm1, Wh1, bh1):
    del node_ids
    pad_e = E_PAD - E
    src_p = jnp.concatenate(
        [edge_index[0], jnp.zeros((pad_e,), jnp.int32)]).reshape(
            NW, K_CHUNKS, CHUNK)
    dst_p = jnp.concatenate(
        [edge_index[1], jnp.full((pad_e,), N, jnp.int32)]).reshape(
            NW, K_CHUNKS, CHUNK)
    ea_p = jnp.concatenate(
        [edge_attr, jnp.zeros((pad_e, D_EDGE), jnp.float32)])

    h = x
    for Wm, bm, Wh, bh in ((Wm0, bm0, Wh0, bh0), (Wm1, bm1, Wh1, bh1)):
        d = h.shape[1]
        h_pad = jnp.concatenate([h, jnp.zeros((N_PAD - N, d), jnp.float32)])
        a_t, b_t = _node_proj(h_pad, Wm[:d], Wm[d:2 * d])
        c_t = _edge_proj(ea_p, Wm[2 * d:], bm.reshape(1, MSG))
        parts = _edge_phase(a_t, b_t, c_t, src_p, dst_p)
        h = _update(parts, h, Wh[:MSG], Wh[MSG:], bh.reshape(1, HID))
    return h


# speedup vs baseline: 3.5747x; 3.5747x over previous
import functools

import jax
import jax.numpy as jnp
from jax import lax
from jax.experimental import pallas as pl
from jax.experimental.pallas import tpu as pltpu
from jax.experimental.pallas import tpu_sc as plsc

N = 10000
E = 320000
D_FEAT = 128
D_EDGE = 16
MSG = 64
HID = 128

NC = 2
NS = 16
NW = NC * NS
CHUNK = 128
K_CHUNKS = 79
EPW = CHUNK * K_CHUNKS
E_PAD = NW * EPW
N_PAD = 10240
EBLK = 2048


def _node_proj_body(h_ref, ws_ref, wd_ref, a_ref, b_ref):
    h = h_ref[...]
    a_ref[...] = jnp.dot(h, ws_ref[...], preferred_element_type=jnp.float32)
    b_ref[...] = jnp.dot(h, wd_ref[...], preferred_element_type=jnp.float32)


_node_proj = pl.pallas_call(
    _node_proj_body,
    out_shape=[
        jax.ShapeDtypeStruct((N_PAD, MSG), jnp.float32),
        jax.ShapeDtypeStruct((N_PAD, MSG), jnp.float32),
    ],
)


def _edge_proj_body(e_ref, w_ref, b_ref, c_ref):
    c_ref[...] = (
        jnp.dot(e_ref[...], w_ref[...], preferred_element_type=jnp.float32)
        + b_ref[...]
    )


_edge_proj = pl.pallas_call(
    _edge_proj_body,
    grid=(E_PAD // EBLK,),
    in_specs=[
        pl.BlockSpec((EBLK, D_EDGE), lambda i: (i, 0)),
        pl.BlockSpec((D_EDGE, MSG), lambda i: (0, 0)),
        pl.BlockSpec((1, MSG), lambda i: (0, 0)),
    ],
    out_specs=pl.BlockSpec((EBLK, MSG), lambda i: (i, 0)),
    out_shape=jax.ShapeDtypeStruct((E_PAD, MSG), jnp.float32),
)


def _update_body(p_ref, h_ref, wt_ref, wb_ref, bh_ref, o_ref):
    m_sum = p_ref[0, :N] + p_ref[1, :N]
    o_ref[...] = jnp.maximum(
        jnp.dot(m_sum, wt_ref[...], preferred_element_type=jnp.float32)
        + jnp.dot(h_ref[...], wb_ref[...], preferred_element_type=jnp.float32)
        + bh_ref[...],
        0.0,
    )


_update = pl.pallas_call(
    _update_body,
    out_shape=jax.ShapeDtypeStruct((N, HID), jnp.float32),
)


_mesh = plsc.VectorSubcoreMesh(core_axis_name="c", subcore_axis_name="s")


@functools.partial(
    pl.kernel,
    out_type=jax.ShapeDtypeStruct((NC, N_PAD, MSG), jnp.float32),
    mesh=_mesh,
    compiler_params=pltpu.CompilerParams(use_tc_tiling_on_sc=False),
    scratch_types=[
        pltpu.VMEM((K_CHUNKS, CHUNK), jnp.int32),
        pltpu.VMEM((K_CHUNKS, CHUNK), jnp.int32),
        pltpu.VMEM((CHUNK, MSG), jnp.float32),
        pltpu.VMEM((CHUNK, MSG), jnp.float32),
        pltpu.VMEM((CHUNK, MSG), jnp.float32),
        pltpu.VMEM((640, MSG), jnp.float32),
        pltpu.VMEM_SHARED((N_PAD, MSG), jnp.float32),
        pltpu.SemaphoreType.DMA,
        pltpu.SemaphoreType.DMA,
        pltpu.SemaphoreType.DMA,
    ],
)
def _edge_phase(a_hbm, b_hbm, c_hbm, src_hbm, dst_hbm, out_hbm,
                src_v, dst_v, a_v, b_v, m_v, z_v, acc, sem_a, sem_b, sem_c):
    cid = lax.axis_index("c")
    sid = lax.axis_index("s")
    wid = sid * NC + cid

    zero16 = jnp.zeros((16,), jnp.float32)

    def _zrow(r, _):
        for j in range(MSG // 16):
            z_v[r, pl.ds(j * 16, 16)] = zero16
        return 0

    lax.fori_loop(0, 640, _zrow, 0)
    pltpu.sync_copy(z_v, acc.at[pl.ds(sid * 640, 640)])
    plsc.subcore_barrier()

    pltpu.sync_copy(src_hbm.at[wid], src_v)
    pltpu.sync_copy(dst_hbm.at[wid], dst_v)
    base = wid * EPW

    def _chunk(k, _):
        cp_a = pltpu.async_copy(a_hbm.at[src_v.at[k]], a_v, sem_a)
        cp_b = pltpu.async_copy(b_hbm.at[dst_v.at[k]], b_v, sem_b)
        cp_c = pltpu.async_copy(
            c_hbm.at[pl.ds(base + k * CHUNK, CHUNK)], m_v, sem_c)
        cp_a.wait()
        cp_b.wait()
        cp_c.wait()

        def _row(r, _):
            for j in range(MSG // 16):
                sl = pl.ds(j * 16, 16)
                m_v[r, sl] = jnp.maximum(a_v[r, sl] + b_v[r, sl] + m_v[r, sl],
                                         0.0)
            return 0

        lax.fori_loop(0, CHUNK, _row, 0)
        pltpu.sync_copy(m_v, acc.at[dst_v.at[k]], add=True)
        return 0

    lax.fori_loop(0, K_CHUNKS, _chunk, 0)
    plsc.subcore_barrier()

    pltpu.sync_copy(acc.at[pl.ds(sid * 640, 640)],
                    out_hbm.at[cid, pl.ds(sid * 640, 640)])


def kernel(x, edge_index, edge_attr, node_ids,
           Wm0, bm0, Wh0, bh0, Wm1, bm1, Wh1, bh1):
    del node_ids
    pad_e = E_PAD - E
    src_p = jnp.concatenate(
        [edge_index[0], jnp.zeros((pad_e,), jnp.int32)]).reshape(
            NW, K_CHUNKS, CHUNK)
    dst_p = jnp.concatenate(
        [edge_index[1], jnp.full((pad_e,), N, jnp.int32)]).reshape(
            NW, K_CHUNKS, CHUNK)
    ea_p = jnp.concatenate(
        [edge_attr, jnp.zeros((pad_e, D_EDGE), jnp.float32)])

    h = x
    for Wm, bm, Wh, bh in ((Wm0, bm0, Wh0, bh0), (Wm1, bm1, Wh1, bh1)):
        d = h.shape[1]
        h_pad = jnp.concatenate([h, jnp.zeros((N_PAD - N, d), jnp.float32)])
        a_t, b_t = _node_proj(h_pad, Wm[:d], Wm[d:2 * d])
        c_t = _edge_proj(ea_p, Wm[2 * d:], bm.reshape(1, MSG))
        parts = _edge_phase(a_t, b_t, c_t, src_p, dst_p)
        h = _update(parts, h, Wh[:MSG], Wh[MSG:], bh.reshape(1, HID))
    return h


# speedup vs baseline: 3.9068x; 1.0929x over previous
import functools

import jax
import jax.numpy as jnp
from jax import lax
from jax.experimental import pallas as pl
from jax.experimental.pallas import tpu as pltpu
from jax.experimental.pallas import tpu_sc as plsc

N = 10000
E = 320000
D_FEAT = 128
D_EDGE = 16
MSG = 64
HID = 128

NC = 2
NS = 16
NW = NC * NS
CHUNK = 128
K_CHUNKS = 79
EPW = CHUNK * K_CHUNKS
E_PAD = NW * EPW
N_PAD = 10240
EBLK = 2048


def _node_proj_body(h_ref, ws_ref, wd_ref, a_ref, b_ref):
    h = h_ref[...]
    a_ref[...] = jnp.dot(h, ws_ref[...], preferred_element_type=jnp.float32)
    b_ref[...] = jnp.dot(h, wd_ref[...], preferred_element_type=jnp.float32)


_node_proj = pl.pallas_call(
    _node_proj_body,
    out_shape=[
        jax.ShapeDtypeStruct((N_PAD, MSG), jnp.float32),
        jax.ShapeDtypeStruct((N_PAD, MSG), jnp.float32),
    ],
)


def _edge_proj_body(e_ref, w_ref, b_ref, c_ref):
    c_ref[...] = (
        jnp.dot(e_ref[...], w_ref[...], preferred_element_type=jnp.float32)
        + b_ref[...]
    )


_edge_proj = pl.pallas_call(
    _edge_proj_body,
    grid=(E_PAD // EBLK,),
    in_specs=[
        pl.BlockSpec((EBLK, D_EDGE), lambda i: (i, 0)),
        pl.BlockSpec((D_EDGE, MSG), lambda i: (0, 0)),
        pl.BlockSpec((1, MSG), lambda i: (0, 0)),
    ],
    out_specs=pl.BlockSpec((EBLK, MSG), lambda i: (i, 0)),
    out_shape=jax.ShapeDtypeStruct((E_PAD, MSG), jnp.float32),
)


def _update_body(p_ref, h_ref, wt_ref, wb_ref, bh_ref, o_ref):
    m_sum = p_ref[0, :N] + p_ref[1, :N]
    o_ref[...] = jnp.maximum(
        jnp.dot(m_sum, wt_ref[...], preferred_element_type=jnp.float32)
        + jnp.dot(h_ref[...], wb_ref[...], preferred_element_type=jnp.float32)
        + bh_ref[...],
        0.0,
    )


_update = pl.pallas_call(
    _update_body,
    out_shape=jax.ShapeDtypeStruct((N, HID), jnp.float32),
)


_mesh = plsc.VectorSubcoreMesh(core_axis_name="c", subcore_axis_name="s")


@functools.partial(
    pl.kernel,
    out_type=jax.ShapeDtypeStruct((NC, N_PAD, MSG), jnp.float32),
    mesh=_mesh,
    compiler_params=pltpu.CompilerParams(use_tc_tiling_on_sc=False),
    scratch_types=[
        pltpu.VMEM((K_CHUNKS, CHUNK), jnp.int32),
        pltpu.VMEM((K_CHUNKS, CHUNK), jnp.int32),
        pltpu.VMEM((CHUNK, MSG), jnp.float32),
        pltpu.VMEM((CHUNK, MSG), jnp.float32),
        pltpu.VMEM((CHUNK, MSG), jnp.float32),
        pltpu.VMEM((CHUNK, MSG), jnp.float32),
        pltpu.VMEM((CHUNK, MSG), jnp.float32),
        pltpu.VMEM((CHUNK, MSG), jnp.float32),
        pltpu.VMEM((CHUNK, MSG), jnp.float32),
        pltpu.VMEM((CHUNK, MSG), jnp.float32),
        pltpu.VMEM_SHARED((N_PAD, MSG), jnp.float32),
        pltpu.SemaphoreType.DMA,
        pltpu.SemaphoreType.DMA,
        pltpu.SemaphoreType.DMA,
        pltpu.SemaphoreType.DMA,
        pltpu.SemaphoreType.DMA,
        pltpu.SemaphoreType.DMA,
    ],
)
def _edge_phase(a_hbm, b_hbm, c_hbm, src_hbm, dst_hbm, out_hbm,
                src_v, dst_v, a0, a1, b0, b1, m0, m1, m2, m3, acc,
                si0, si1, ss0, ss1, ss2, ss3):
    cid = lax.axis_index("c")
    sid = lax.axis_index("s")
    wid = sid * NC + cid
    abuf = (a0, a1)
    bbuf = (b0, b1)
    mbuf = (m0, m1, m2, m3)
    si = (si0, si1)
    ss = (ss0, ss1, ss2, ss3)
    dummy = c_hbm.at[pl.ds(0, CHUNK)]

    zero16 = jnp.zeros((16,), jnp.float32)

    def _zrow(r, _):
        for j in range(MSG // 16):
            m0[r, pl.ds(j * 16, 16)] = zero16
        return 0

    lax.fori_loop(0, CHUNK, _zrow, 0)
    for j in range(5):
        pltpu.sync_copy(m0, acc.at[pl.ds(sid * 640 + j * CHUNK, CHUNK)])
    plsc.subcore_barrier()

    pltpu.sync_copy(src_hbm.at[wid], src_v)
    pltpu.sync_copy(dst_hbm.at[wid], dst_v)
    base = wid * EPW

    def fire(k, par, slot):
        @pl.when(k >= 4)
        def _():
            pltpu.make_async_copy(dummy, mbuf[slot], ss[slot]).wait()

        pltpu.async_copy(a_hbm.at[src_v.at[k]], abuf[par], si[par])
        pltpu.async_copy(b_hbm.at[dst_v.at[k]], bbuf[par], si[par])
        pltpu.async_copy(c_hbm.at[pl.ds(base + k * CHUNK, CHUNK)],
                         mbuf[slot], si[par])

    def proc(k, par, slot):
        a_v, b_v, m_v = abuf[par], bbuf[par], mbuf[slot]
        pltpu.make_async_copy(dummy, a_v, si[par]).wait()
        pltpu.make_async_copy(dummy, b_v, si[par]).wait()
        pltpu.make_async_copy(dummy, m_v, si[par]).wait()

        def _row(r4, _):
            for rr in range(4):
                r = r4 * 4 + rr
                for j in range(MSG // 16):
                    sl = pl.ds(j * 16, 16)
                    m_v[r, sl] = jnp.maximum(
                        a_v[r, sl] + b_v[r, sl] + m_v[r, sl], 0.0)
            return 0

        lax.fori_loop(0, CHUNK // 4, _row, 0)
        pltpu.async_copy(m_v, acc.at[dst_v.at[k]], ss[slot], add=True)

    fire(0, 0, 0)

    def _quad(i, _):
        k = 4 * i
        fire(k + 1, 1, 1)
        proc(k, 0, 0)
        fire(k + 2, 0, 2)
        proc(k + 1, 1, 1)
        fire(k + 3, 1, 3)
        proc(k + 2, 0, 2)
        fire(k + 4, 0, 0)
        proc(k + 3, 1, 3)
        return 0

    lax.fori_loop(0, (K_CHUNKS - 3) // 4, _quad, 0)
    fire(77, 1, 1)
    proc(76, 0, 0)
    fire(78, 0, 2)
    proc(77, 1, 1)
    proc(78, 0, 2)
    for j in range(4):
        pltpu.make_async_copy(dummy, mbuf[j], ss[j]).wait()
    plsc.subcore_barrier()

    pltpu.sync_copy(acc.at[pl.ds(sid * 640, 640)],
                    out_hbm.at[cid, pl.ds(sid * 640, 640)])


def kernel(x, edge_index, edge_attr, node_ids,
           Wm0, bm0, Wh0, bh0, Wm1, bm1, Wh1, bh1):
    del node_ids
    pad_e = E_PAD - E
    src_p = jnp.concatenate(
        [edge_index[0], jnp.zeros((pad_e,), jnp.int32)]).reshape(
            NW, K_CHUNKS, CHUNK)
    dst_p = jnp.concatenate(
        [edge_index[1], jnp.full((pad_e,), N, jnp.int32)]).reshape(
            NW, K_CHUNKS, CHUNK)
    ea_p = jnp.concatenate(
        [edge_attr, jnp.zeros((pad_e, D_EDGE), jnp.float32)])

    h = x
    for Wm, bm, Wh, bh in ((Wm0, bm0, Wh0, bh0), (Wm1, bm1, Wh1, bh1)):
        d = h.shape[1]
        h_pad = jnp.concatenate([h, jnp.zeros((N_PAD - N, d), jnp.float32)])
        a_t, b_t = _node_proj(h_pad, Wm[:d], Wm[d:2 * d])
        c_t = _edge_proj(ea_p, Wm[2 * d:], bm.reshape(1, MSG))
        parts = _edge_phase(a_t, b_t, c_t, src_p, dst_p)
        h = _update(parts, h, Wh[:MSG], Wh[MSG:], bh.reshape(1, HID))
    return h
